# Initial kernel scaffold; baseline (speedup 1.0000x reference)
import jax
import jax.numpy as jnp
from jax.experimental import pallas as pl


def kernel(x, control_points, derivatives):
    raise NotImplementedError("write your pallas kernel here")


# speedup vs baseline: 474.5972x; 474.5972x over previous
import functools

import jax
import jax.numpy as jnp
from jax import lax
from jax.experimental import pallas as pl
from jax.experimental.pallas import tpu as pltpu
from jax.experimental.pallas import tpu_sc as plsc

N = 4194304
NUM_SEGMENTS = 64
_INFO = plsc.get_sparse_core_info()
NC = _INFO.num_cores
NS = _INFO.num_subcores
NW = NC * NS
PER_W = N // NW
CH = 8192
NCH = PER_W // CH
L = 16


def _spline_body(x_hbm, cp_hbm, dv_hbm, out_hbm,
                 cp_v, dv_v, tbl_v, xb0, xb1, yb0, yb1,
                 sem_in0, sem_in1, sem_out0, sem_out1):
    wid = lax.axis_index("s") * NC + lax.axis_index("c")
    base = wid * PER_W

    xb = (xb0, xb1)
    yb = (yb0, yb1)
    sem_in = (sem_in0, sem_in1)
    sem_out = (sem_out0, sem_out1)

    cp_in = [None, None]
    cp_in[0] = pltpu.async_copy(x_hbm.at[pl.ds(base, CH)], xb[0], sem_in[0])

    pltpu.sync_copy(cp_hbm, cp_v)
    pltpu.sync_copy(dv_hbm, dv_v)
    for j in range(NUM_SEGMENTS // L):
        off = j * L
        y0 = cp_v[pl.ds(off, L)]
        y1 = cp_v[pl.ds(off + 1, L)]
        d0 = dv_v[pl.ds(off, L)]
        d1 = dv_v[pl.ds(off + 1, L)]
        dy = y1 - y0
        tbl_v[pl.ds(off, L)] = y0
        tbl_v[pl.ds(64 + off, L)] = d0
        tbl_v[pl.ds(128 + off, L)] = (48.0 * dy - 2.0 * d0 - d1) * 16.0
        tbl_v[pl.ds(192 + off, L)] = (-32.0 * dy + d0 + d1) * 256.0

    cp_out = [None] * NCH

    for ch in range(NCH):
        b = ch % 2
        if ch + 1 < NCH:
            cp_in[(ch + 1) % 2] = pltpu.async_copy(
                x_hbm.at[pl.ds(base + (ch + 1) * CH, CH)],
                xb[(ch + 1) % 2], sem_in[(ch + 1) % 2])
        cp_in[b].wait()
        if ch >= 2:
            cp_out[ch - 2].wait()

        xref = xb[b]
        yref = yb[b]

        def vec_step(i, _):
            off = pl.multiple_of(i * L, L)
            xv = xref[pl.ds(off, L)]
            seg = (xv * 16.0 + 32.0).astype(jnp.int32)
            seg = jnp.minimum(jnp.maximum(seg, 0), NUM_SEGMENTS - 1)
            t = xv - (seg.astype(jnp.float32) * 0.0625 - 2.0)
            ca = plsc.load_gather(tbl_v, [seg])
            cb = plsc.load_gather(tbl_v, [seg + 64])
            cc = plsc.load_gather(tbl_v, [seg + 128])
            cd = plsc.load_gather(tbl_v, [seg + 192])
            yref[pl.ds(off, L)] = ca + t * (cb + t * (cc + t * cd))
            return 0

        lax.fori_loop(0, CH // L, vec_step, 0)

        cp_out[ch] = pltpu.async_copy(
            yb[b], out_hbm.at[pl.ds(base + ch * CH, CH)], sem_out[b])

    cp_out[NCH - 2].wait()
    cp_out[NCH - 1].wait()


_spline_sc = pl.kernel(
    _spline_body,
    out_type=jax.ShapeDtypeStruct((N,), jnp.float32),
    mesh=plsc.VectorSubcoreMesh(core_axis_name="c", subcore_axis_name="s"),
    scratch_types=[
        pltpu.VMEM((NUM_SEGMENTS + 1,), jnp.float32),
        pltpu.VMEM((NUM_SEGMENTS + 1,), jnp.float32),
        pltpu.VMEM((4 * NUM_SEGMENTS,), jnp.float32),
        pltpu.VMEM((CH,), jnp.float32),
        pltpu.VMEM((CH,), jnp.float32),
        pltpu.VMEM((CH,), jnp.float32),
        pltpu.VMEM((CH,), jnp.float32),
        pltpu.SemaphoreType.DMA,
        pltpu.SemaphoreType.DMA,
        pltpu.SemaphoreType.DMA,
        pltpu.SemaphoreType.DMA,
    ],
    compiler_params=pltpu.CompilerParams(needs_layout_passes=False),
)


@jax.jit
def kernel(x, control_points, derivatives):
    y = _spline_sc(x, control_points[:, 0], derivatives[:, 0])
    return y[:, None]


# speedup vs baseline: 963.4847x; 2.0301x over previous
import functools

import jax
import jax.numpy as jnp
from jax import lax
from jax.experimental import pallas as pl
from jax.experimental.pallas import tpu as pltpu
from jax.experimental.pallas import tpu_sc as plsc

N = 4194304
NUM_SEGMENTS = 64
_INFO = plsc.get_sparse_core_info()
NC = _INFO.num_cores
NS = _INFO.num_subcores
NW = NC * NS
PER_W = N // NW
CH = 16384
NCH = PER_W // CH
L = 16


def _spline_body(x_hbm, cp_hbm, dv_hbm, out_hbm,
                 cp_v, dv_v, tbl_v, xb0, xb1, yb0, yb1,
                 sem_in0, sem_in1, sem_out0, sem_out1):
    wid = lax.axis_index("s") * NC + lax.axis_index("c")
    base = wid * PER_W

    xb = (xb0, xb1)
    yb = (yb0, yb1)
    sem_in = (sem_in0, sem_in1)
    sem_out = (sem_out0, sem_out1)

    cp_in = [None, None]
    cp_in[0] = pltpu.async_copy(x_hbm.at[pl.ds(base, CH)], xb[0], sem_in[0])

    pltpu.sync_copy(cp_hbm, cp_v)
    pltpu.sync_copy(dv_hbm, dv_v)
    for j in range(NUM_SEGMENTS // L):
        off = j * L
        y0 = cp_v[pl.ds(off, L)]
        y1 = cp_v[pl.ds(off + 1, L)]
        d0 = dv_v[pl.ds(off, L)]
        d1 = dv_v[pl.ds(off + 1, L)]
        dy = y1 - y0
        tbl_v[pl.ds(off, L)] = y0
        tbl_v[pl.ds(64 + off, L)] = d0
        tbl_v[pl.ds(128 + off, L)] = (48.0 * dy - 2.0 * d0 - d1) * 16.0
        tbl_v[pl.ds(192 + off, L)] = (-32.0 * dy + d0 + d1) * 256.0

    cp_out = [None] * NCH

    for ch in range(NCH):
        b = ch % 2
        if ch + 1 < NCH:
            cp_in[(ch + 1) % 2] = pltpu.async_copy(
                x_hbm.at[pl.ds(base + (ch + 1) * CH, CH)],
                xb[(ch + 1) % 2], sem_in[(ch + 1) % 2])
        cp_in[b].wait()
        if ch >= 2:
            cp_out[ch - 2].wait()

        xref = xb[b]
        yref = yb[b]

        @plsc.parallel_loop(0, CH, step=L, unroll=8)
        def _vec_step(off):
            xv = xref[pl.ds(off, L)]
            seg = (xv * 16.0 + 32.0).astype(jnp.int32)
            seg = jnp.minimum(jnp.maximum(seg, 0), NUM_SEGMENTS - 1)
            t = xv - (seg.astype(jnp.float32) * 0.0625 - 2.0)
            ca = plsc.load_gather(tbl_v, [seg])
            cb = plsc.load_gather(tbl_v, [seg + 64])
            cc = plsc.load_gather(tbl_v, [seg + 128])
            cd = plsc.load_gather(tbl_v, [seg + 192])
            yref[pl.ds(off, L)] = ca + t * (cb + t * (cc + t * cd))

        cp_out[ch] = pltpu.async_copy(
            yb[b], out_hbm.at[pl.ds(base + ch * CH, CH)], sem_out[b])

    cp_out[NCH - 2].wait()
    cp_out[NCH - 1].wait()


_spline_sc = pl.kernel(
    _spline_body,
    out_type=jax.ShapeDtypeStruct((N,), jnp.float32),
    mesh=plsc.VectorSubcoreMesh(core_axis_name="c", subcore_axis_name="s"),
    scratch_types=[
        pltpu.VMEM((NUM_SEGMENTS + 1,), jnp.float32),
        pltpu.VMEM((NUM_SEGMENTS + 1,), jnp.float32),
        pltpu.VMEM((4 * NUM_SEGMENTS,), jnp.float32),
        pltpu.VMEM((CH,), jnp.float32),
        pltpu.VMEM((CH,), jnp.float32),
        pltpu.VMEM((CH,), jnp.float32),
        pltpu.VMEM((CH,), jnp.float32),
        pltpu.SemaphoreType.DMA,
        pltpu.SemaphoreType.DMA,
        pltpu.SemaphoreType.DMA,
        pltpu.SemaphoreType.DMA,
    ],
    compiler_params=pltpu.CompilerParams(needs_layout_passes=False),
)


@jax.jit
def kernel(x, control_points, derivatives):
    y = _spline_sc(x, control_points[:, 0], derivatives[:, 0])
    return y[:, None]
